# baseline (speedup 1.0000x reference)
import functools

import jax
import jax.numpy as jnp
from jax import lax
from jax.experimental import pallas as pl
from jax.experimental.pallas import tpu as pltpu
from jax.experimental.pallas import tpu_sc as plsc

NATOMS = 10000
EDGES = 320000
DIM = 128
NF = 128
NG = 50
CUT = 5.0
NB = 10
HID = 64

NCORES = 2
NSUB = 16
NWORK = NCORES * NSUB
LANES = 16

EW = EDGES // NWORK
ECH = 2000
CH = 40
NCH = EW // CH
NBUF = 2
NIB = 4
NQ = (NCH - 2) // 4
RPT = 624
ZR = 48

_LN2 = 0.6931471805599453

_mesh = plsc.VectorSubcoreMesh(core_axis_name="c", subcore_axis_name="s")
_sc_params = pltpu.CompilerParams(needs_layout_passes=False)


def _ssp(x):
    return jnp.logaddexp(x, 0.0) - _LN2


@functools.partial(
    pl.kernel,
    out_type=jax.ShapeDtypeStruct((EDGES,), jnp.float32),
    mesh=_mesh,
    scratch_types=[
        pltpu.VMEM((NATOMS,), jnp.float32),
        pltpu.VMEM((NATOMS,), jnp.float32),
        pltpu.VMEM((NATOMS,), jnp.float32),
        pltpu.VMEM((ECH,), jnp.int32),
        pltpu.VMEM((ECH,), jnp.int32),
        pltpu.VMEM((ECH,), jnp.float32),
        pltpu.SemaphoreType.DMA,
    ],
    compiler_params=_sc_params,
)
def _dist(xs_hbm, ys_hbm, zs_hbm, a0_hbm, a1_hbm, d2_hbm,
          xs_v, ys_v, zs_v, i0_v, i1_v, d2_v, sd):
    cid = lax.axis_index("c")
    sid = lax.axis_index("s")
    base = (cid * NSUB + sid) * EW
    pltpu.async_copy(xs_hbm, xs_v, sd)
    pltpu.async_copy(ys_hbm, ys_v, sd)
    pltpu.async_copy(zs_hbm, zs_v, sd)
    pltpu.make_async_copy(xs_hbm, xs_v, sd).wait()
    pltpu.make_async_copy(ys_hbm, ys_v, sd).wait()
    pltpu.make_async_copy(zs_hbm, zs_v, sd).wait()

    @pl.loop(0, EW // ECH)
    def _chunk(g):
        off = base + g * ECH
        pltpu.async_copy(a0_hbm.at[pl.ds(off, ECH)], i0_v, sd)
        pltpu.async_copy(a1_hbm.at[pl.ds(off, ECH)], i1_v, sd)
        pltpu.make_async_copy(a0_hbm.at[pl.ds(off, ECH)], i0_v, sd).wait()
        pltpu.make_async_copy(a1_hbm.at[pl.ds(off, ECH)], i1_v, sd).wait()

        @pl.loop(0, ECH // LANES)
        def _vec(k):
            s = pl.ds(k * LANES, LANES)
            i0 = i0_v[s]
            i1 = i1_v[s]
            dx = plsc.load_gather(xs_v, [i0]) - plsc.load_gather(xs_v, [i1])
            dy = plsc.load_gather(ys_v, [i0]) - plsc.load_gather(ys_v, [i1])
            dz = plsc.load_gather(zs_v, [i0]) - plsc.load_gather(zs_v, [i1])
            d2_v[s] = dx * dx + dy * dy + dz * dz + 1e-12

        pltpu.sync_copy(d2_v, d2_hbm.at[pl.ds(off, ECH)])


@functools.partial(
    pl.kernel,
    out_type=jax.ShapeDtypeStruct((2, NATOMS, NF), jnp.float32),
    mesh=_mesh,
    scratch_types=[
        pltpu.VMEM((NIB, 2 * CH), jnp.int32),
        pltpu.VMEM((NIB, 2 * CH), jnp.int32),
        pltpu.VMEM((NBUF, 2 * CH, NF), jnp.float32),
        pltpu.VMEM((NBUF, CH, NF), jnp.float32),
        pltpu.VMEM((ZR, NF), jnp.float32),
        pltpu.VMEM_SHARED((NATOMS, NF), jnp.float32),
        pltpu.SemaphoreType.DMA,
        pltpu.SemaphoreType.DMA,
        pltpu.SemaphoreType.DMA,
        pltpu.SemaphoreType.DMA,
        pltpu.SemaphoreType.DMA,
        pltpu.SemaphoreType.DMA,
        pltpu.SemaphoreType.DMA,
        pltpu.SemaphoreType.DMA,
    ],
    compiler_params=_sc_params,
)
def _msg(r2_hbm, w_hbm, ag_hbm, as_hbm, out_hbm,
         ig_v, is_v, rows_v, w_v, zero_v, agg_sh,
         sin0, sin1, sout0, sout1, sidx0, sidx1, sidx2, sidx3):
    cid = lax.axis_index("c")
    sid = lax.axis_index("s")
    base = (cid * NSUB + sid) * EW
    sin = (sin0, sin1)
    sout = (sout0, sout1)
    sidx = (sidx0, sidx1, sidx2, sidx3)

    @pl.loop(0, ZR)
    def _zfill(i):
        for j in range(NF // LANES):
            zero_v[i, pl.ds(j * LANES, LANES)] = jnp.zeros((LANES,), jnp.float32)

    for t in range(RPT // ZR):
        pltpu.async_copy(zero_v, agg_sh.at[pl.ds(sid * RPT + t * ZR, ZR)], sin0)

    @pl.when(sid == NSUB - 1)
    def _ztail():
        pltpu.async_copy(zero_v.at[pl.ds(0, 16)],
                         agg_sh.at[pl.ds(NSUB * RPT, 16)], sin0)

    for t in range(RPT // ZR):
        pltpu.make_async_copy(
            zero_v, agg_sh.at[pl.ds(sid * RPT + t * ZR, ZR)], sin0).wait()

    @pl.when(sid == NSUB - 1)
    def _ztailw():
        pltpu.make_async_copy(zero_v.at[pl.ds(0, 16)],
                              agg_sh.at[pl.ds(NSUB * RPT, 16)], sin0).wait()

    plsc.subcore_barrier()

    def issue_idx(g, ib):
        off = 2 * (base + g * CH)
        pltpu.async_copy(ag_hbm.at[pl.ds(off, 2 * CH)], ig_v.at[ib], sidx[ib])
        pltpu.async_copy(as_hbm.at[pl.ds(off, 2 * CH)], is_v.at[ib], sidx[ib])

    def wait_idx(g, ib):
        off = 2 * (base + g * CH)
        pltpu.make_async_copy(ag_hbm.at[pl.ds(off, 2 * CH)], ig_v.at[ib], sidx[ib]).wait()
        pltpu.make_async_copy(as_hbm.at[pl.ds(off, 2 * CH)], is_v.at[ib], sidx[ib]).wait()

    def issue_in(g, rb, ib):
        off = base + g * CH
        pltpu.async_copy(r2_hbm.at[ig_v.at[ib]], rows_v.at[rb], sin[rb])
        pltpu.async_copy(w_hbm.at[pl.ds(off, CH)], w_v.at[rb], sin[rb])

    def wait_in(g, rb, ib):
        off = base + g * CH
        pltpu.make_async_copy(r2_hbm.at[ig_v.at[ib]], rows_v.at[rb], sin[rb]).wait()
        pltpu.make_async_copy(w_hbm.at[pl.ds(off, CH)], w_v.at[rb], sin[rb]).wait()

    def compute(rb):
        @pl.loop(0, CH)
        def _mul(e):
            for j in range(NF // LANES):
                s = pl.ds(j * LANES, LANES)
                w = w_v[rb, e, s]
                rows_v[rb, 2 * e, s] = rows_v[rb, 2 * e, s] * w
                rows_v[rb, 2 * e + 1, s] = rows_v[rb, 2 * e + 1, s] * w

    def issue_out(rb, ib):
        pltpu.async_copy(rows_v.at[rb], agg_sh.at[is_v.at[ib]], sout[rb], add=True)

    def wait_out(rb, ib):
        pltpu.make_async_copy(rows_v.at[rb], agg_sh.at[is_v.at[ib]], sout[rb]).wait()

    issue_idx(0, 0)
    issue_idx(1, 1)
    wait_idx(0, 0)
    issue_in(0, 0, 0)
    issue_idx(2, 2)

    @pl.loop(0, NQ)
    def _quad(q):
        for k in range(4):
            g = q * 4 + k
            rb = k % 2
            ib = k
            ibn = (k + 1) % 4
            ibp = (k + 3) % 4

            if k == 0:
                @pl.when(q > 0)
                def _w_prev():
                    wait_out(1 - rb, 3)
            else:
                wait_out(1 - rb, k - 1)

            wait_idx(g + 1, ibn)
            issue_in(g + 1, 1 - rb, ibn)

            if k == 3:
                @pl.when(q < NQ - 1)
                def _idx_nxt():
                    issue_idx(g + 3, ibp)
            else:
                issue_idx(g + 3, ibp)

            wait_in(g, rb, ib)
            compute(rb)
            issue_out(rb, ib)

    gt = NQ * 4
    wait_out(1, 3)
    wait_idx(gt + 1, 1)
    issue_in(gt + 1, 1, 1)
    wait_in(gt, 0, 0)
    compute(0)
    issue_out(0, 0)

    wait_out(0, 0)
    wait_in(gt + 1, 1, 1)
    compute(1)
    issue_out(1, 1)

    wait_out(1, 1)
    plsc.subcore_barrier()
    for t in range(RPT // ZR):
        r0 = sid * RPT + t * ZR
        pltpu.async_copy(agg_sh.at[pl.ds(r0, ZR)], out_hbm.at[cid, pl.ds(r0, ZR)],
                         sin0)

    @pl.when(sid == NSUB - 1)
    def _otail():
        pltpu.async_copy(agg_sh.at[pl.ds(NSUB * RPT, 16)],
                         out_hbm.at[cid, pl.ds(NSUB * RPT, 16)], sin0)

    for t in range(RPT // ZR):
        r0 = sid * RPT + t * ZR
        pltpu.make_async_copy(agg_sh.at[pl.ds(r0, ZR)],
                              out_hbm.at[cid, pl.ds(r0, ZR)], sin0).wait()

    @pl.when(sid == NSUB - 1)
    def _otailw():
        pltpu.make_async_copy(agg_sh.at[pl.ds(NSUB * RPT, 16)],
                              out_hbm.at[cid, pl.ds(NSUB * RPT, 16)], sin0).wait()


_BE = 2000


def _filters_body(d2_ref, wf1, bf1, wf2, bf2, out):
    d = jnp.sqrt(d2_ref[:])
    col = lax.broadcasted_iota(jnp.int32, (_BE, NG), 1).astype(jnp.float32)
    width = CUT / (NG - 1)
    z = d * (1.0 / width) - col
    g = jnp.exp(-0.5 * z * z)
    t = _ssp(jnp.dot(g, wf1[:], preferred_element_type=jnp.float32) + bf1[:])
    out[:] = jnp.dot(t, wf2[:], preferred_element_type=jnp.float32) + bf2[:]


def _filters(d2, wf1, bf1, wf2, bf2):
    n = EDGES // _BE
    full = lambda arr: pl.BlockSpec(arr.shape, lambda i: (0,) * arr.ndim)
    return pl.pallas_call(
        _filters_body,
        grid=(n,),
        in_specs=[pl.BlockSpec((_BE, 1), lambda i: (i, 0)),
                  full(wf1), full(bf1), full(wf2), full(bf2)],
        out_specs=pl.BlockSpec((_BE, NF), lambda i: (i, 0)),
        out_shape=jax.ShapeDtypeStruct((EDGES, NF), jnp.float32),
    )(d2, wf1, bf1, wf2, bf2)


_AB = 2000


def _embed_body(r_ref, emb_ref, win_ref, bin_ref, h_ref, r2_ref):
    idx = r_ref[:]
    col = lax.broadcasted_iota(jnp.int32, (_AB, 100), 1)
    onehot = (idx == col).astype(jnp.float32)
    h = jnp.dot(onehot, emb_ref[:], preferred_element_type=jnp.float32)
    h_ref[:] = h
    r2_ref[:] = jnp.dot(h, win_ref[:], preferred_element_type=jnp.float32) + bin_ref[:]


def _embed(r, emb, win, bin_):
    full = lambda arr: pl.BlockSpec(arr.shape, lambda i: (0,) * arr.ndim)
    return pl.pallas_call(
        _embed_body,
        grid=(NATOMS // _AB,),
        in_specs=[pl.BlockSpec((_AB, 1), lambda i: (i, 0)),
                  full(emb), full(win), full(bin_)],
        out_specs=[pl.BlockSpec((_AB, DIM), lambda i: (i, 0))] * 2,
        out_shape=[jax.ShapeDtypeStruct((NATOMS, DIM), jnp.float32)] * 2,
    )(r, emb, win, bin_)


def _mid_body(p_ref, h_ref, wo1_ref, bo1_ref, wo2_ref, bo2_ref,
              win_ref, bin_ref, h_out, r2_out):
    agg = p_ref[0] + p_ref[1]
    t = _ssp(jnp.dot(agg, wo1_ref[:], preferred_element_type=jnp.float32) + bo1_ref[:])
    dr = jnp.dot(t, wo2_ref[:], preferred_element_type=jnp.float32) + bo2_ref[:]
    hn = h_ref[:] + dr
    h_out[:] = hn
    r2_out[:] = jnp.dot(hn, win_ref[:], preferred_element_type=jnp.float32) + bin_ref[:]


def _mid(parts, h, wo1, bo1, wo2, bo2, win, bin_):
    full = lambda arr: pl.BlockSpec(arr.shape, lambda i: (0,) * arr.ndim)
    blk = pl.BlockSpec((_AB, DIM), lambda i: (i, 0))
    pblk = pl.BlockSpec((2, _AB, DIM), lambda i: (0, i, 0))
    return pl.pallas_call(
        _mid_body,
        grid=(NATOMS // _AB,),
        in_specs=[pblk, blk, full(wo1), full(bo1), full(wo2), full(bo2),
                  full(win), full(bin_)],
        out_specs=[blk, blk],
        out_shape=[jax.ShapeDtypeStruct((NATOMS, DIM), jnp.float32)] * 2,
    )(parts, h, wo1, bo1, wo2, bo2, win, bin_)


_MB = NATOMS // NB


def _end_body(p_ref, h_ref, wo1_ref, bo1_ref, wo2_ref, bo2_ref,
              wr1_ref, br1_ref, wr2_ref, br2_ref, e_out):
    agg = p_ref[0] + p_ref[1]
    t = _ssp(jnp.dot(agg, wo1_ref[:], preferred_element_type=jnp.float32) + bo1_ref[:])
    dr = jnp.dot(t, wo2_ref[:], preferred_element_type=jnp.float32) + bo2_ref[:]
    hn = h_ref[:] + dr
    e1 = _ssp(jnp.dot(hn, wr1_ref[:], preferred_element_type=jnp.float32) + br1_ref[:])
    e2 = _ssp(jnp.dot(e1, wr2_ref[:], preferred_element_type=jnp.float32) + br2_ref[:])
    e_out[:] = jnp.sum(e2).reshape(1, 1, 1)


def _end(parts, h, wo1, bo1, wo2, bo2, wr1, br1, wr2, br2):
    full = lambda arr: pl.BlockSpec(arr.shape, lambda i: (0,) * arr.ndim)
    blk = pl.BlockSpec((_MB, DIM), lambda i: (i, 0))
    pblk = pl.BlockSpec((2, _MB, DIM), lambda i: (0, i, 0))
    return pl.pallas_call(
        _end_body,
        grid=(NB,),
        in_specs=[pblk, blk, full(wo1), full(bo1), full(wo2), full(bo2),
                  full(wr1), full(br1), full(wr2), full(br2)],
        out_specs=pl.BlockSpec((1, 1, 1), lambda i: (i, 0, 0)),
        out_shape=jax.ShapeDtypeStruct((NB, 1, 1), jnp.float32),
    )(parts, h, wo1, bo1, wo2, bo2, wr1, br1, wr2, br2).reshape(NB, 1)


def kernel(r, xyz, N, a, params):
    a0 = jnp.asarray(a[:, 0], jnp.int32)
    a1 = jnp.asarray(a[:, 1], jnp.int32)
    ag = jnp.stack([a0, a1], axis=1).reshape(2 * EDGES)
    asw = jnp.stack([a1, a0], axis=1).reshape(2 * EDGES)
    xs = jnp.asarray(xyz[:, 0])
    ys = jnp.asarray(xyz[:, 1])
    zs = jnp.asarray(xyz[:, 2])

    d2 = _dist(xs, ys, zs, a0, a1)

    convs = params['convs']
    d2c = d2.reshape(EDGES, 1)

    def filt(c):
        cp = convs[c]
        return _filters(d2c, cp['Wf1'], cp['bf1'].reshape(1, NF),
                        cp['Wf2'], cp['bf2'].reshape(1, NF))

    h, r2 = _embed(jnp.asarray(r, jnp.int32), params['embed'],
                   convs[0]['Win'], convs[0]['bin'].reshape(1, NF))

    ws = [filt(0), filt(1), filt(2)]

    for c in range(3):
        cp = convs[c]
        parts = _msg(r2, ws[c], ag, asw)
        if c < 2:
            nxt = convs[c + 1]
            h, r2 = _mid(parts, h,
                         cp['Wo1'], cp['bo1'].reshape(1, DIM),
                         cp['Wo2'], cp['bo2'].reshape(1, DIM),
                         nxt['Win'], nxt['bin'].reshape(1, NF))
        else:
            energy = _end(parts, h,
                          cp['Wo1'], cp['bo1'].reshape(1, DIM),
                          cp['Wo2'], cp['bo2'].reshape(1, DIM),
                          params['Wr1'], params['br1'].reshape(1, HID),
                          params['Wr2'], params['br2'].reshape(1, 1))
    return energy


# speedup vs baseline: 1.7118x; 1.7118x over previous
import functools

import jax
import jax.numpy as jnp
from jax import lax
from jax.experimental import pallas as pl
from jax.experimental.pallas import tpu as pltpu
from jax.experimental.pallas import tpu_sc as plsc

NATOMS = 10000
EDGES = 320000
DIM = 128
NF = 128
NG = 50
CUT = 5.0
NB = 10
HID = 64

NCORES = 2
NSUB = 16
NWORK = NCORES * NSUB
LANES = 16

EW = EDGES // NWORK
ECH = 2000
CH = 40
NCH = EW // CH
NBUF = 2
NIB = 4
NQ = (NCH - 2) // 4
RPT = 624
ZR = 48

_LN2 = 0.6931471805599453

_mesh = plsc.VectorSubcoreMesh(core_axis_name="c", subcore_axis_name="s")
_sc_params = pltpu.CompilerParams(needs_layout_passes=False)


def _ssp(x):
    return jnp.logaddexp(x, 0.0) - _LN2


@functools.partial(
    pl.kernel,
    out_type=jax.ShapeDtypeStruct((EDGES,), jnp.float32),
    mesh=_mesh,
    scratch_types=[
        pltpu.VMEM((NATOMS,), jnp.float32),
        pltpu.VMEM((NATOMS,), jnp.float32),
        pltpu.VMEM((NATOMS,), jnp.float32),
        pltpu.VMEM((ECH,), jnp.int32),
        pltpu.VMEM((ECH,), jnp.int32),
        pltpu.VMEM((ECH,), jnp.float32),
        pltpu.SemaphoreType.DMA,
    ],
    compiler_params=_sc_params,
)
def _dist(xs_hbm, ys_hbm, zs_hbm, a0_hbm, a1_hbm, d2_hbm,
          xs_v, ys_v, zs_v, i0_v, i1_v, d2_v, sd):
    cid = lax.axis_index("c")
    sid = lax.axis_index("s")
    base = (cid * NSUB + sid) * EW
    pltpu.async_copy(xs_hbm, xs_v, sd)
    pltpu.async_copy(ys_hbm, ys_v, sd)
    pltpu.async_copy(zs_hbm, zs_v, sd)
    pltpu.make_async_copy(xs_hbm, xs_v, sd).wait()
    pltpu.make_async_copy(ys_hbm, ys_v, sd).wait()
    pltpu.make_async_copy(zs_hbm, zs_v, sd).wait()

    @pl.loop(0, EW // ECH)
    def _chunk(g):
        off = base + g * ECH
        pltpu.async_copy(a0_hbm.at[pl.ds(off, ECH)], i0_v, sd)
        pltpu.async_copy(a1_hbm.at[pl.ds(off, ECH)], i1_v, sd)
        pltpu.make_async_copy(a0_hbm.at[pl.ds(off, ECH)], i0_v, sd).wait()
        pltpu.make_async_copy(a1_hbm.at[pl.ds(off, ECH)], i1_v, sd).wait()

        @pl.loop(0, ECH // LANES)
        def _vec(k):
            s = pl.ds(k * LANES, LANES)
            i0 = i0_v[s]
            i1 = i1_v[s]
            dx = plsc.load_gather(xs_v, [i0]) - plsc.load_gather(xs_v, [i1])
            dy = plsc.load_gather(ys_v, [i0]) - plsc.load_gather(ys_v, [i1])
            dz = plsc.load_gather(zs_v, [i0]) - plsc.load_gather(zs_v, [i1])
            d2_v[s] = dx * dx + dy * dy + dz * dz + 1e-12

        pltpu.sync_copy(d2_v, d2_hbm.at[pl.ds(off, ECH)])


@functools.partial(
    pl.kernel,
    out_type=jax.ShapeDtypeStruct((2, NATOMS, NF), jnp.float32),
    mesh=_mesh,
    scratch_types=[
        pltpu.VMEM((NIB, CH), jnp.int32),
        pltpu.VMEM((NIB, CH), jnp.int32),
        pltpu.VMEM((NBUF, CH, NF), jnp.float32),
        pltpu.VMEM((NBUF, CH, NF), jnp.float32),
        pltpu.VMEM((NBUF, CH, NF), jnp.float32),
        pltpu.VMEM((ZR, NF), jnp.float32),
        pltpu.VMEM_SHARED((NATOMS, NF), jnp.float32),
        pltpu.SemaphoreType.DMA,
        pltpu.SemaphoreType.DMA,
        pltpu.SemaphoreType.DMA,
        pltpu.SemaphoreType.DMA,
        pltpu.SemaphoreType.DMA,
        pltpu.SemaphoreType.DMA,
        pltpu.SemaphoreType.DMA,
        pltpu.SemaphoreType.DMA,
    ],
    compiler_params=_sc_params,
)
def _msg(r2_hbm, w_hbm, a0_hbm, a1_hbm, out_hbm,
         i0_v, i1_v, rows0_v, rows1_v, w_v, zero_v, agg_sh,
         sin0, sin1, sout0, sout1, sidx0, sidx1, sidx2, sidx3):
    cid = lax.axis_index("c")
    sid = lax.axis_index("s")
    base = (cid * NSUB + sid) * EW
    sin = (sin0, sin1)
    sout = (sout0, sout1)
    sidx = (sidx0, sidx1, sidx2, sidx3)

    @pl.loop(0, ZR)
    def _zfill(i):
        for j in range(NF // LANES):
            zero_v[i, pl.ds(j * LANES, LANES)] = jnp.zeros((LANES,), jnp.float32)

    for t in range(RPT // ZR):
        pltpu.async_copy(zero_v, agg_sh.at[pl.ds(sid * RPT + t * ZR, ZR)], sin0)

    @pl.when(sid == NSUB - 1)
    def _ztail():
        pltpu.async_copy(zero_v.at[pl.ds(0, 16)],
                         agg_sh.at[pl.ds(NSUB * RPT, 16)], sin0)

    for t in range(RPT // ZR):
        pltpu.make_async_copy(
            zero_v, agg_sh.at[pl.ds(sid * RPT + t * ZR, ZR)], sin0).wait()

    @pl.when(sid == NSUB - 1)
    def _ztailw():
        pltpu.make_async_copy(zero_v.at[pl.ds(0, 16)],
                              agg_sh.at[pl.ds(NSUB * RPT, 16)], sin0).wait()

    plsc.subcore_barrier()

    def issue_idx(g, ib):
        off = base + g * CH
        pltpu.async_copy(a0_hbm.at[pl.ds(off, CH)], i0_v.at[ib], sidx[ib])
        pltpu.async_copy(a1_hbm.at[pl.ds(off, CH)], i1_v.at[ib], sidx[ib])

    def wait_idx(g, ib):
        off = base + g * CH
        pltpu.make_async_copy(a0_hbm.at[pl.ds(off, CH)], i0_v.at[ib], sidx[ib]).wait()
        pltpu.make_async_copy(a1_hbm.at[pl.ds(off, CH)], i1_v.at[ib], sidx[ib]).wait()

    def issue_in(g, rb, ib):
        off = base + g * CH
        pltpu.async_copy(r2_hbm.at[i0_v.at[ib]], rows0_v.at[rb], sin[rb])
        pltpu.async_copy(r2_hbm.at[i1_v.at[ib]], rows1_v.at[rb], sin[rb])
        pltpu.async_copy(w_hbm.at[pl.ds(off, CH)], w_v.at[rb], sin[rb])

    def wait_in(g, rb, ib):
        off = base + g * CH
        pltpu.make_async_copy(r2_hbm.at[i0_v.at[ib]], rows0_v.at[rb], sin[rb]).wait()
        pltpu.make_async_copy(r2_hbm.at[i1_v.at[ib]], rows1_v.at[rb], sin[rb]).wait()
        pltpu.make_async_copy(w_hbm.at[pl.ds(off, CH)], w_v.at[rb], sin[rb]).wait()

    def compute(rb):
        @pl.loop(0, CH)
        def _mul(e):
            for j in range(NF // LANES):
                s = pl.ds(j * LANES, LANES)
                w = w_v[rb, e, s]
                rows0_v[rb, e, s] = rows0_v[rb, e, s] * w
                rows1_v[rb, e, s] = rows1_v[rb, e, s] * w

    def issue_out(rb, ib):
        pltpu.async_copy(rows0_v.at[rb], agg_sh.at[i1_v.at[ib]], sout[rb], add=True)
        pltpu.async_copy(rows1_v.at[rb], agg_sh.at[i0_v.at[ib]], sout[rb], add=True)

    def wait_out(rb, ib):
        pltpu.make_async_copy(rows0_v.at[rb], agg_sh.at[i1_v.at[ib]], sout[rb]).wait()
        pltpu.make_async_copy(rows1_v.at[rb], agg_sh.at[i0_v.at[ib]], sout[rb]).wait()

    issue_idx(0, 0)
    issue_idx(1, 1)
    wait_idx(0, 0)
    issue_in(0, 0, 0)
    issue_idx(2, 2)

    @pl.loop(0, NQ)
    def _quad(q):
        for k in range(4):
            g = q * 4 + k
            rb = k % 2
            ib = k
            ibn = (k + 1) % 4
            ibp = (k + 3) % 4

            if k == 0:
                @pl.when(q > 0)
                def _w_prev():
                    wait_out(1 - rb, 3)
            else:
                wait_out(1 - rb, k - 1)

            wait_idx(g + 1, ibn)
            issue_in(g + 1, 1 - rb, ibn)

            if k == 3:
                @pl.when(q < NQ - 1)
                def _idx_nxt():
                    issue_idx(g + 3, ibp)
            else:
                issue_idx(g + 3, ibp)

            wait_in(g, rb, ib)
            compute(rb)
            issue_out(rb, ib)

    gt = NQ * 4
    wait_out(1, 3)
    wait_idx(gt + 1, 1)
    issue_in(gt + 1, 1, 1)
    wait_in(gt, 0, 0)
    compute(0)
    issue_out(0, 0)

    wait_out(0, 0)
    wait_in(gt + 1, 1, 1)
    compute(1)
    issue_out(1, 1)

    wait_out(1, 1)
    plsc.subcore_barrier()
    for t in range(RPT // ZR):
        r0 = sid * RPT + t * ZR
        pltpu.async_copy(agg_sh.at[pl.ds(r0, ZR)], out_hbm.at[cid, pl.ds(r0, ZR)],
                         sin0)

    @pl.when(sid == NSUB - 1)
    def _otail():
        pltpu.async_copy(agg_sh.at[pl.ds(NSUB * RPT, 16)],
                         out_hbm.at[cid, pl.ds(NSUB * RPT, 16)], sin0)

    for t in range(RPT // ZR):
        r0 = sid * RPT + t * ZR
        pltpu.make_async_copy(agg_sh.at[pl.ds(r0, ZR)],
                              out_hbm.at[cid, pl.ds(r0, ZR)], sin0).wait()

    @pl.when(sid == NSUB - 1)
    def _otailw():
        pltpu.make_async_copy(agg_sh.at[pl.ds(NSUB * RPT, 16)],
                              out_hbm.at[cid, pl.ds(NSUB * RPT, 16)], sin0).wait()


_BE = 2000


def _filters_body(d2_ref, wf1, bf1, wf2, bf2, out):
    d = jnp.sqrt(d2_ref[:])
    col = lax.broadcasted_iota(jnp.int32, (_BE, NG), 1).astype(jnp.float32)
    width = CUT / (NG - 1)
    z = d * (1.0 / width) - col
    g = jnp.exp(-0.5 * z * z)
    t = _ssp(jnp.dot(g, wf1[:], preferred_element_type=jnp.float32) + bf1[:])
    out[:] = jnp.dot(t, wf2[:], preferred_element_type=jnp.float32) + bf2[:]


def _filters(d2, wf1, bf1, wf2, bf2):
    n = EDGES // _BE
    full = lambda arr: pl.BlockSpec(arr.shape, lambda i: (0,) * arr.ndim)
    return pl.pallas_call(
        _filters_body,
        grid=(n,),
        in_specs=[pl.BlockSpec((_BE, 1), lambda i: (i, 0)),
                  full(wf1), full(bf1), full(wf2), full(bf2)],
        out_specs=pl.BlockSpec((_BE, NF), lambda i: (i, 0)),
        out_shape=jax.ShapeDtypeStruct((EDGES, NF), jnp.float32),
    )(d2, wf1, bf1, wf2, bf2)


_AB = 2000


def _embed_body(r_ref, emb_ref, win_ref, bin_ref, h_ref, r2_ref):
    idx = r_ref[:]
    col = lax.broadcasted_iota(jnp.int32, (_AB, 100), 1)
    onehot = (idx == col).astype(jnp.float32)
    h = jnp.dot(onehot, emb_ref[:], preferred_element_type=jnp.float32)
    h_ref[:] = h
    r2_ref[:] = jnp.dot(h, win_ref[:], preferred_element_type=jnp.float32) + bin_ref[:]


def _embed(r, emb, win, bin_):
    full = lambda arr: pl.BlockSpec(arr.shape, lambda i: (0,) * arr.ndim)
    return pl.pallas_call(
        _embed_body,
        grid=(NATOMS // _AB,),
        in_specs=[pl.BlockSpec((_AB, 1), lambda i: (i, 0)),
                  full(emb), full(win), full(bin_)],
        out_specs=[pl.BlockSpec((_AB, DIM), lambda i: (i, 0))] * 2,
        out_shape=[jax.ShapeDtypeStruct((NATOMS, DIM), jnp.float32)] * 2,
    )(r, emb, win, bin_)


def _mid_body(p_ref, h_ref, wo1_ref, bo1_ref, wo2_ref, bo2_ref,
              win_ref, bin_ref, h_out, r2_out):
    agg = p_ref[0] + p_ref[1]
    t = _ssp(jnp.dot(agg, wo1_ref[:], preferred_element_type=jnp.float32) + bo1_ref[:])
    dr = jnp.dot(t, wo2_ref[:], preferred_element_type=jnp.float32) + bo2_ref[:]
    hn = h_ref[:] + dr
    h_out[:] = hn
    r2_out[:] = jnp.dot(hn, win_ref[:], preferred_element_type=jnp.float32) + bin_ref[:]


def _mid(parts, h, wo1, bo1, wo2, bo2, win, bin_):
    full = lambda arr: pl.BlockSpec(arr.shape, lambda i: (0,) * arr.ndim)
    blk = pl.BlockSpec((_AB, DIM), lambda i: (i, 0))
    pblk = pl.BlockSpec((2, _AB, DIM), lambda i: (0, i, 0))
    return pl.pallas_call(
        _mid_body,
        grid=(NATOMS // _AB,),
        in_specs=[pblk, blk, full(wo1), full(bo1), full(wo2), full(bo2),
                  full(win), full(bin_)],
        out_specs=[blk, blk],
        out_shape=[jax.ShapeDtypeStruct((NATOMS, DIM), jnp.float32)] * 2,
    )(parts, h, wo1, bo1, wo2, bo2, win, bin_)


_MB = NATOMS // NB


def _end_body(p_ref, h_ref, wo1_ref, bo1_ref, wo2_ref, bo2_ref,
              wr1_ref, br1_ref, wr2_ref, br2_ref, e_out):
    agg = p_ref[0] + p_ref[1]
    t = _ssp(jnp.dot(agg, wo1_ref[:], preferred_element_type=jnp.float32) + bo1_ref[:])
    dr = jnp.dot(t, wo2_ref[:], preferred_element_type=jnp.float32) + bo2_ref[:]
    hn = h_ref[:] + dr
    e1 = _ssp(jnp.dot(hn, wr1_ref[:], preferred_element_type=jnp.float32) + br1_ref[:])
    e2 = _ssp(jnp.dot(e1, wr2_ref[:], preferred_element_type=jnp.float32) + br2_ref[:])
    e_out[:] = jnp.sum(e2).reshape(1, 1, 1)


def _end(parts, h, wo1, bo1, wo2, bo2, wr1, br1, wr2, br2):
    full = lambda arr: pl.BlockSpec(arr.shape, lambda i: (0,) * arr.ndim)
    blk = pl.BlockSpec((_MB, DIM), lambda i: (i, 0))
    pblk = pl.BlockSpec((2, _MB, DIM), lambda i: (0, i, 0))
    return pl.pallas_call(
        _end_body,
        grid=(NB,),
        in_specs=[pblk, blk, full(wo1), full(bo1), full(wo2), full(bo2),
                  full(wr1), full(br1), full(wr2), full(br2)],
        out_specs=pl.BlockSpec((1, 1, 1), lambda i: (i, 0, 0)),
        out_shape=jax.ShapeDtypeStruct((NB, 1, 1), jnp.float32),
    )(parts, h, wo1, bo1, wo2, bo2, wr1, br1, wr2, br2).reshape(NB, 1)


def kernel(r, xyz, N, a, params):
    a0 = jnp.asarray(a[:, 0], jnp.int32)
    a1 = jnp.asarray(a[:, 1], jnp.int32)
    xs = jnp.asarray(xyz[:, 0])
    ys = jnp.asarray(xyz[:, 1])
    zs = jnp.asarray(xyz[:, 2])

    d2 = _dist(xs, ys, zs, a0, a1)

    convs = params['convs']
    d2c = d2.reshape(EDGES, 1)

    def filt(c):
        cp = convs[c]
        return _filters(d2c, cp['Wf1'], cp['bf1'].reshape(1, NF),
                        cp['Wf2'], cp['bf2'].reshape(1, NF))

    h, r2 = _embed(jnp.asarray(r, jnp.int32), params['embed'],
                   convs[0]['Win'], convs[0]['bin'].reshape(1, NF))

    ws = [filt(0), filt(1), filt(2)]

    for c in range(3):
        cp = convs[c]
        parts = _msg(r2, ws[c], a0, a1)
        if c < 2:
            nxt = convs[c + 1]
            h, r2 = _mid(parts, h,
                         cp['Wo1'], cp['bo1'].reshape(1, DIM),
                         cp['Wo2'], cp['bo2'].reshape(1, DIM),
                         nxt['Win'], nxt['bin'].reshape(1, NF))
        else:
            energy = _end(parts, h,
                          cp['Wo1'], cp['bo1'].reshape(1, DIM),
                          cp['Wo2'], cp['bo2'].reshape(1, DIM),
                          params['Wr1'], params['br1'].reshape(1, HID),
                          params['Wr2'], params['br2'].reshape(1, 1))
    return energy


# speedup vs baseline: 1.7957x; 1.0490x over previous
import functools

import jax
import jax.numpy as jnp
from jax import lax
from jax.experimental import pallas as pl
from jax.experimental.pallas import tpu as pltpu
from jax.experimental.pallas import tpu_sc as plsc

NATOMS = 10000
EDGES = 320000
DIM = 128
NF = 128
NG = 50
CUT = 5.0
NB = 10
HID = 64

NCORES = 2
NSUB = 16
NWORK = NCORES * NSUB
LANES = 16

EW = EDGES // NWORK
ECH = 2000
CH = 40
NCH = EW // CH
NBUF = 3
NIB = 6
NQ = (NCH - 4) // 6
RPT = 624
ZR = 16

_LN2 = 0.6931471805599453

_mesh = plsc.VectorSubcoreMesh(core_axis_name="c", subcore_axis_name="s")
_sc_params = pltpu.CompilerParams(needs_layout_passes=False)


def _ssp(x):
    return jnp.logaddexp(x, 0.0) - _LN2


@functools.partial(
    pl.kernel,
    out_type=jax.ShapeDtypeStruct((EDGES,), jnp.float32),
    mesh=_mesh,
    scratch_types=[
        pltpu.VMEM((NATOMS,), jnp.float32),
        pltpu.VMEM((NATOMS,), jnp.float32),
        pltpu.VMEM((NATOMS,), jnp.float32),
        pltpu.VMEM((ECH,), jnp.int32),
        pltpu.VMEM((ECH,), jnp.int32),
        pltpu.VMEM((ECH,), jnp.float32),
        pltpu.SemaphoreType.DMA,
    ],
    compiler_params=_sc_params,
)
def _dist(xs_hbm, ys_hbm, zs_hbm, a0_hbm, a1_hbm, d2_hbm,
          xs_v, ys_v, zs_v, i0_v, i1_v, d2_v, sd):
    cid = lax.axis_index("c")
    sid = lax.axis_index("s")
    base = (cid * NSUB + sid) * EW
    pltpu.async_copy(xs_hbm, xs_v, sd)
    pltpu.async_copy(ys_hbm, ys_v, sd)
    pltpu.async_copy(zs_hbm, zs_v, sd)
    pltpu.make_async_copy(xs_hbm, xs_v, sd).wait()
    pltpu.make_async_copy(ys_hbm, ys_v, sd).wait()
    pltpu.make_async_copy(zs_hbm, zs_v, sd).wait()

    @pl.loop(0, EW // ECH)
    def _chunk(g):
        off = base + g * ECH
        pltpu.async_copy(a0_hbm.at[pl.ds(off, ECH)], i0_v, sd)
        pltpu.async_copy(a1_hbm.at[pl.ds(off, ECH)], i1_v, sd)
        pltpu.make_async_copy(a0_hbm.at[pl.ds(off, ECH)], i0_v, sd).wait()
        pltpu.make_async_copy(a1_hbm.at[pl.ds(off, ECH)], i1_v, sd).wait()

        @pl.loop(0, ECH // LANES)
        def _vec(k):
            s = pl.ds(k * LANES, LANES)
            i0 = i0_v[s]
            i1 = i1_v[s]
            dx = plsc.load_gather(xs_v, [i0]) - plsc.load_gather(xs_v, [i1])
            dy = plsc.load_gather(ys_v, [i0]) - plsc.load_gather(ys_v, [i1])
            dz = plsc.load_gather(zs_v, [i0]) - plsc.load_gather(zs_v, [i1])
            d2_v[s] = dx * dx + dy * dy + dz * dz + 1e-12

        pltpu.sync_copy(d2_v, d2_hbm.at[pl.ds(off, ECH)])


@functools.partial(
    pl.kernel,
    out_type=jax.ShapeDtypeStruct((2, NATOMS, NF), jnp.float32),
    mesh=_mesh,
    scratch_types=[
        pltpu.VMEM((NIB, CH), jnp.int32),
        pltpu.VMEM((NIB, CH), jnp.int32),
        pltpu.VMEM((NBUF, CH, NF), jnp.float32),
        pltpu.VMEM((NBUF, CH, NF), jnp.float32),
        pltpu.VMEM((NBUF, CH, NF), jnp.float32),
        pltpu.VMEM((ZR, NF), jnp.float32),
        pltpu.VMEM_SHARED((NATOMS, NF), jnp.float32),
    ] + [pltpu.SemaphoreType.DMA] * 12,
    compiler_params=_sc_params,
)
def _msg(r2_hbm, w_hbm, a0_hbm, a1_hbm, out_hbm,
         i0_v, i1_v, rows0_v, rows1_v, w_v, zero_v, agg_sh,
         sin0, sin1, sin2, sout0, sout1, sout2,
         sidx0, sidx1, sidx2, sidx3, sidx4, sidx5):
    cid = lax.axis_index("c")
    sid = lax.axis_index("s")
    base = (cid * NSUB + sid) * EW
    sin = (sin0, sin1, sin2)
    sout = (sout0, sout1, sout2)
    sidx = (sidx0, sidx1, sidx2, sidx3, sidx4, sidx5)

    @pl.loop(0, ZR)
    def _zfill(i):
        for j in range(NF // LANES):
            zero_v[i, pl.ds(j * LANES, LANES)] = jnp.zeros((LANES,), jnp.float32)

    for t in range(RPT // ZR):
        pltpu.async_copy(zero_v, agg_sh.at[pl.ds(sid * RPT + t * ZR, ZR)], sin0)

    @pl.when(sid == NSUB - 1)
    def _ztail():
        pltpu.async_copy(zero_v.at[pl.ds(0, 16)],
                         agg_sh.at[pl.ds(NSUB * RPT, 16)], sin0)

    for t in range(RPT // ZR):
        pltpu.make_async_copy(
            zero_v, agg_sh.at[pl.ds(sid * RPT + t * ZR, ZR)], sin0).wait()

    @pl.when(sid == NSUB - 1)
    def _ztailw():
        pltpu.make_async_copy(zero_v.at[pl.ds(0, 16)],
                              agg_sh.at[pl.ds(NSUB * RPT, 16)], sin0).wait()

    plsc.subcore_barrier()

    def issue_idx(g, ib):
        off = base + g * CH
        pltpu.async_copy(a0_hbm.at[pl.ds(off, CH)], i0_v.at[ib], sidx[ib])
        pltpu.async_copy(a1_hbm.at[pl.ds(off, CH)], i1_v.at[ib], sidx[ib])

    def wait_idx(g, ib):
        off = base + g * CH
        pltpu.make_async_copy(a0_hbm.at[pl.ds(off, CH)], i0_v.at[ib], sidx[ib]).wait()
        pltpu.make_async_copy(a1_hbm.at[pl.ds(off, CH)], i1_v.at[ib], sidx[ib]).wait()

    def issue_in(g, rb, ib):
        off = base + g * CH
        pltpu.async_copy(r2_hbm.at[i0_v.at[ib]], rows0_v.at[rb], sin[rb])
        pltpu.async_copy(r2_hbm.at[i1_v.at[ib]], rows1_v.at[rb], sin[rb])
        pltpu.async_copy(w_hbm.at[pl.ds(off, CH)], w_v.at[rb], sin[rb])

    def wait_in(g, rb, ib):
        off = base + g * CH
        pltpu.make_async_copy(r2_hbm.at[i0_v.at[ib]], rows0_v.at[rb], sin[rb]).wait()
        pltpu.make_async_copy(r2_hbm.at[i1_v.at[ib]], rows1_v.at[rb], sin[rb]).wait()
        pltpu.make_async_copy(w_hbm.at[pl.ds(off, CH)], w_v.at[rb], sin[rb]).wait()

    def compute(rb):
        @pl.loop(0, CH)
        def _mul(e):
            for j in range(NF // LANES):
                s = pl.ds(j * LANES, LANES)
                w = w_v[rb, e, s]
                rows0_v[rb, e, s] = rows0_v[rb, e, s] * w
                rows1_v[rb, e, s] = rows1_v[rb, e, s] * w

    def issue_out(rb, ib):
        pltpu.async_copy(rows0_v.at[rb], agg_sh.at[i1_v.at[ib]], sout[rb], add=True)
        pltpu.async_copy(rows1_v.at[rb], agg_sh.at[i0_v.at[ib]], sout[rb], add=True)

    def wait_out(rb, ib):
        pltpu.make_async_copy(rows0_v.at[rb], agg_sh.at[i1_v.at[ib]], sout[rb]).wait()
        pltpu.make_async_copy(rows1_v.at[rb], agg_sh.at[i0_v.at[ib]], sout[rb]).wait()

    issue_idx(0, 0)
    issue_idx(1, 1)
    issue_idx(2, 2)
    wait_idx(0, 0)
    issue_in(0, 0, 0)
    issue_idx(3, 3)

    @pl.loop(0, NQ)
    def _sext(t):
        for k in range(6):
            g = t * 6 + k
            rb = k % 3
            ib = k
            rbn = (k + 1) % 3
            ibn = (k + 1) % 6
            rb2 = (k - 2) % 3
            ib2 = (k - 2) % 6
            ibp = (k + 4) % 6

            if k < 2:
                @pl.when(t > 0)
                def _w_prev():
                    wait_out(rb2, ib2)
            else:
                wait_out(rb2, ib2)

            wait_idx(g + 1, ibn)
            issue_in(g + 1, rbn, ibn)
            issue_idx(g + 4, ibp)

            wait_in(g, rb, ib)
            compute(rb)
            issue_out(rb, ib)

    gt = NQ * 6
    wait_idx(gt + 1, 1)
    wait_out(1, 4)
    issue_in(gt + 1, 1, 1)
    wait_in(gt, 0, 0)
    compute(0)
    issue_out(0, 0)

    wait_idx(gt + 2, 2)
    wait_out(2, 5)
    issue_in(gt + 2, 2, 2)
    wait_in(gt + 1, 1, 1)
    compute(1)
    issue_out(1, 1)

    wait_idx(gt + 3, 3)
    wait_out(0, 0)
    issue_in(gt + 3, 0, 3)
    wait_in(gt + 2, 2, 2)
    compute(2)
    issue_out(2, 2)

    wait_out(1, 1)
    wait_in(gt + 3, 0, 3)
    compute(0)
    issue_out(0, 3)

    wait_out(2, 2)
    wait_out(0, 3)
    plsc.subcore_barrier()
    for t in range(RPT // ZR):
        r0 = sid * RPT + t * ZR
        pltpu.async_copy(agg_sh.at[pl.ds(r0, ZR)], out_hbm.at[cid, pl.ds(r0, ZR)],
                         sin0)

    @pl.when(sid == NSUB - 1)
    def _otail():
        pltpu.async_copy(agg_sh.at[pl.ds(NSUB * RPT, 16)],
                         out_hbm.at[cid, pl.ds(NSUB * RPT, 16)], sin0)

    for t in range(RPT // ZR):
        r0 = sid * RPT + t * ZR
        pltpu.make_async_copy(agg_sh.at[pl.ds(r0, ZR)],
                              out_hbm.at[cid, pl.ds(r0, ZR)], sin0).wait()

    @pl.when(sid == NSUB - 1)
    def _otailw():
        pltpu.make_async_copy(agg_sh.at[pl.ds(NSUB * RPT, 16)],
                              out_hbm.at[cid, pl.ds(NSUB * RPT, 16)], sin0).wait()


_BE = 2000


def _filters_body(d2_ref, wf1, bf1, wf2, bf2, out):
    d = jnp.sqrt(d2_ref[:])
    col = lax.broadcasted_iota(jnp.int32, (_BE, NG), 1).astype(jnp.float32)
    width = CUT / (NG - 1)
    z = d * (1.0 / width) - col
    g = jnp.exp(-0.5 * z * z)
    t = _ssp(jnp.dot(g, wf1[:], preferred_element_type=jnp.float32) + bf1[:])
    out[:] = jnp.dot(t, wf2[:], preferred_element_type=jnp.float32) + bf2[:]


def _filters(d2, wf1, bf1, wf2, bf2):
    n = EDGES // _BE
    full = lambda arr: pl.BlockSpec(arr.shape, lambda i: (0,) * arr.ndim)
    return pl.pallas_call(
        _filters_body,
        grid=(n,),
        in_specs=[pl.BlockSpec((_BE, 1), lambda i: (i, 0)),
                  full(wf1), full(bf1), full(wf2), full(bf2)],
        out_specs=pl.BlockSpec((_BE, NF), lambda i: (i, 0)),
        out_shape=jax.ShapeDtypeStruct((EDGES, NF), jnp.float32),
    )(d2, wf1, bf1, wf2, bf2)


_AB = 2000


def _embed_body(r_ref, emb_ref, win_ref, bin_ref, h_ref, r2_ref):
    idx = r_ref[:]
    col = lax.broadcasted_iota(jnp.int32, (_AB, 100), 1)
    onehot = (idx == col).astype(jnp.float32)
    h = jnp.dot(onehot, emb_ref[:], preferred_element_type=jnp.float32)
    h_ref[:] = h
    r2_ref[:] = jnp.dot(h, win_ref[:], preferred_element_type=jnp.float32) + bin_ref[:]


def _embed(r, emb, win, bin_):
    full = lambda arr: pl.BlockSpec(arr.shape, lambda i: (0,) * arr.ndim)
    return pl.pallas_call(
        _embed_body,
        grid=(NATOMS // _AB,),
        in_specs=[pl.BlockSpec((_AB, 1), lambda i: (i, 0)),
                  full(emb), full(win), full(bin_)],
        out_specs=[pl.BlockSpec((_AB, DIM), lambda i: (i, 0))] * 2,
        out_shape=[jax.ShapeDtypeStruct((NATOMS, DIM), jnp.float32)] * 2,
    )(r, emb, win, bin_)


def _mid_body(p_ref, h_ref, wo1_ref, bo1_ref, wo2_ref, bo2_ref,
              win_ref, bin_ref, h_out, r2_out):
    agg = p_ref[0] + p_ref[1]
    t = _ssp(jnp.dot(agg, wo1_ref[:], preferred_element_type=jnp.float32) + bo1_ref[:])
    dr = jnp.dot(t, wo2_ref[:], preferred_element_type=jnp.float32) + bo2_ref[:]
    hn = h_ref[:] + dr
    h_out[:] = hn
    r2_out[:] = jnp.dot(hn, win_ref[:], preferred_element_type=jnp.float32) + bin_ref[:]


def _mid(parts, h, wo1, bo1, wo2, bo2, win, bin_):
    full = lambda arr: pl.BlockSpec(arr.shape, lambda i: (0,) * arr.ndim)
    blk = pl.BlockSpec((_AB, DIM), lambda i: (i, 0))
    pblk = pl.BlockSpec((2, _AB, DIM), lambda i: (0, i, 0))
    return pl.pallas_call(
        _mid_body,
        grid=(NATOMS // _AB,),
        in_specs=[pblk, blk, full(wo1), full(bo1), full(wo2), full(bo2),
                  full(win), full(bin_)],
        out_specs=[blk, blk],
        out_shape=[jax.ShapeDtypeStruct((NATOMS, DIM), jnp.float32)] * 2,
    )(parts, h, wo1, bo1, wo2, bo2, win, bin_)


_MB = NATOMS // NB


def _end_body(p_ref, h_ref, wo1_ref, bo1_ref, wo2_ref, bo2_ref,
              wr1_ref, br1_ref, wr2_ref, br2_ref, e_out):
    agg = p_ref[0] + p_ref[1]
    t = _ssp(jnp.dot(agg, wo1_ref[:], preferred_element_type=jnp.float32) + bo1_ref[:])
    dr = jnp.dot(t, wo2_ref[:], preferred_element_type=jnp.float32) + bo2_ref[:]
    hn = h_ref[:] + dr
    e1 = _ssp(jnp.dot(hn, wr1_ref[:], preferred_element_type=jnp.float32) + br1_ref[:])
    e2 = _ssp(jnp.dot(e1, wr2_ref[:], preferred_element_type=jnp.float32) + br2_ref[:])
    e_out[:] = jnp.sum(e2).reshape(1, 1, 1)


def _end(parts, h, wo1, bo1, wo2, bo2, wr1, br1, wr2, br2):
    full = lambda arr: pl.BlockSpec(arr.shape, lambda i: (0,) * arr.ndim)
    blk = pl.BlockSpec((_MB, DIM), lambda i: (i, 0))
    pblk = pl.BlockSpec((2, _MB, DIM), lambda i: (0, i, 0))
    return pl.pallas_call(
        _end_body,
        grid=(NB,),
        in_specs=[pblk, blk, full(wo1), full(bo1), full(wo2), full(bo2),
                  full(wr1), full(br1), full(wr2), full(br2)],
        out_specs=pl.BlockSpec((1, 1, 1), lambda i: (i, 0, 0)),
        out_shape=jax.ShapeDtypeStruct((NB, 1, 1), jnp.float32),
    )(parts, h, wo1, bo1, wo2, bo2, wr1, br1, wr2, br2).reshape(NB, 1)


def kernel(r, xyz, N, a, params):
    a0 = jnp.asarray(a[:, 0], jnp.int32)
    a1 = jnp.asarray(a[:, 1], jnp.int32)
    xs = jnp.asarray(xyz[:, 0])
    ys = jnp.asarray(xyz[:, 1])
    zs = jnp.asarray(xyz[:, 2])

    d2 = _dist(xs, ys, zs, a0, a1)

    convs = params['convs']
    d2c = d2.reshape(EDGES, 1)

    def filt(c):
        cp = convs[c]
        return _filters(d2c, cp['Wf1'], cp['bf1'].reshape(1, NF),
                        cp['Wf2'], cp['bf2'].reshape(1, NF))

    h, r2 = _embed(jnp.asarray(r, jnp.int32), params['embed'],
                   convs[0]['Win'], convs[0]['bin'].reshape(1, NF))

    ws = [filt(0), filt(1), filt(2)]

    for c in range(3):
        cp = convs[c]
        parts = _msg(r2, ws[c], a0, a1)
        if c < 2:
            nxt = convs[c + 1]
            h, r2 = _mid(parts, h,
                         cp['Wo1'], cp['bo1'].reshape(1, DIM),
                         cp['Wo2'], cp['bo2'].reshape(1, DIM),
                         nxt['Win'], nxt['bin'].reshape(1, NF))
        else:
            energy = _end(parts, h,
                          cp['Wo1'], cp['bo1'].reshape(1, DIM),
                          cp['Wo2'], cp['bo2'].reshape(1, DIM),
                          params['Wr1'], params['br1'].reshape(1, HID),
                          params['Wr2'], params['br2'].reshape(1, 1))
    return energy
